# baseline (speedup 1.0000x reference)
import functools

import jax
import jax.numpy as jnp
from jax import lax
from jax.experimental import pallas as pl
from jax.experimental.pallas import tpu as pltpu
from jax.experimental.pallas import tpu_sc as plsc

_N = 10000
_E = 320000
_D = 128
_G = 128
_C = 16

_NC = 2
_NS = 16
_NW = _NC * _NS
_EPW = _E // _NW
_CH = 80
_NCHUNK = _EPW // _CH
_RMAIN = 624
_CPR = 48
_RREM = _N - _NS * _RMAIN


def _bn_relu(h, g, be):
    m = jnp.mean(h, axis=0, keepdims=True)
    v = jnp.mean((h - m) ** 2, axis=0, keepdims=True)
    return jnp.maximum((h - m) * lax.rsqrt(v + 1e-5) * g + be, 0.0)


def _mlp_block(h, W1, b1, g1, be1, W2, b2, g2, be2):
    h = jnp.dot(h, W1, preferred_element_type=jnp.float32) + b1
    h = _bn_relu(h, g1, be1)
    h = jnp.dot(h, W2, preferred_element_type=jnp.float32) + b2
    return _bn_relu(h, g2, be2)


def _pool_mats(batch_row):
    onehot = (batch_row == lax.broadcasted_iota(jnp.int32, (_G, _N), 0)
              ).astype(jnp.float32)
    counts = jnp.sum(onehot, axis=1, keepdims=True)
    return onehot, counts


def _tc1_body(x_ref, W1, b1, g1, be1, W2, b2, g2, be2, x1_out):
    x1_out[...] = _mlp_block(x_ref[...], W1[...], b1[...], g1[...], be1[...],
                             W2[...], b2[...], g2[...], be2[...])


def _tcp_body(x1_ref, batch_ref, l0W, l0b, out0_out):
    onehot, counts = _pool_mats(batch_ref[...])
    pooled = jnp.dot(onehot, x1_ref[...], preferred_element_type=jnp.float32)
    out0_out[...] = (jnp.dot(pooled, l0W[...], preferred_element_type=jnp.float32)
                     + counts * l0b[...])


def _tc2_body(agg_ref, batch_ref, out0_ref, W1, b1, g1, be1,
              W2, b2, g2, be2, l1W, l1b, out_ref):
    z = agg_ref[0] + agg_ref[1]
    x2 = _mlp_block(z, W1[...], b1[...], g1[...], be1[...],
                    W2[...], b2[...], g2[...], be2[...])
    onehot, counts = _pool_mats(batch_ref[...])
    pooled = jnp.dot(onehot, x2, preferred_element_type=jnp.float32)
    out = (out0_ref[...]
           + jnp.dot(pooled, l1W[...], preferred_element_type=jnp.float32)
           + counts * l1b[...])
    mx = jnp.max(out, axis=-1, keepdims=True)
    e = jnp.exp(out - mx)
    out_ref[...] = out - mx - jnp.log(jnp.sum(e, axis=-1, keepdims=True))


def _sc_agg_body(edges_hbm, x1_hbm, out_hbm,
                 dst_all, sidx_a, sidx_b, rows_a, rows_b, acc,
                 sem_d, sem_sa, sem_sb, sem_a, sem_b):
    c = lax.axis_index("c")
    s = lax.axis_index("s")
    w = s * _NC + c

    dd = pltpu.async_copy(edges_hbm.at[1, w], dst_all, sem_d)
    pltpu.async_copy(edges_hbm.at[0, w, 0], sidx_a, sem_sa)
    pltpu.async_copy(edges_hbm.at[0, w, 1], sidx_b, sem_sb)

    row0 = s * _RMAIN

    @pl.when(c == 0)
    def _():
        pltpu.sync_copy(x1_hbm.at[pl.ds(row0, _RMAIN)],
                        acc.at[pl.ds(row0, _RMAIN)])

        @pl.when(s == _NS - 1)
        def _():
            pltpu.sync_copy(x1_hbm.at[pl.ds(_NS * _RMAIN, _RREM)],
                            acc.at[pl.ds(_NS * _RMAIN, _RREM)])

    @pl.when(c == 1)
    def _():
        def zstore(i, _):
            r = i // (_D // 16)
            col = (i % (_D // 16)) * 16
            rows_a[r, pl.ds(col, 16)] = jnp.zeros((16,), jnp.float32)
            return 0
        lax.fori_loop(0, _CPR * _D // 16, zstore, 0)
        for b in range(_RMAIN // _CPR):
            pltpu.sync_copy(rows_a.at[pl.ds(0, _CPR)],
                            acc.at[pl.ds(row0 + b * _CPR, _CPR)])

        @pl.when(s == _NS - 1)
        def _():
            pltpu.sync_copy(rows_a.at[pl.ds(0, _RREM)],
                            acc.at[pl.ds(_NS * _RMAIN, _RREM)])

    pltpu.make_async_copy(edges_hbm.at[0, w, 1], sidx_b, sem_sb).wait()
    pltpu.async_copy(x1_hbm.at[sidx_b], rows_b, sem_b)
    pltpu.make_async_copy(edges_hbm.at[0, w, 0], sidx_a, sem_sa).wait()
    pltpu.async_copy(x1_hbm.at[sidx_a], rows_a, sem_a)
    dd.wait()
    plsc.subcore_barrier()


    def half(j, sidx, rows, sem_s, sem_r):
        pltpu.make_async_copy(x1_hbm.at[sidx], rows, sem_r).wait()
        jn = jnp.minimum(j + 2, _NCHUNK - 1)
        pltpu.async_copy(edges_hbm.at[0, w, jn], sidx, sem_s)
        pltpu.sync_copy(rows, acc.at[dst_all.at[j]], add=True)
        pltpu.make_async_copy(edges_hbm.at[0, w, jn], sidx, sem_s).wait()
        pltpu.async_copy(x1_hbm.at[sidx], rows, sem_r)

    def step(i, _):
        j = 2 * i
        half(j, sidx_a, rows_a, sem_sa, sem_a)
        half(j + 1, sidx_b, rows_b, sem_sb, sem_b)
        return 0
    lax.fori_loop(0, (_NCHUNK - 1) // 2, step, 0)

    pltpu.make_async_copy(x1_hbm.at[sidx_a], rows_a, sem_a).wait()
    pltpu.sync_copy(rows_a, acc.at[dst_all.at[_NCHUNK - 1]], add=True)
    pltpu.make_async_copy(x1_hbm.at[sidx_b], rows_b, sem_b).wait()
    plsc.subcore_barrier()

    pltpu.sync_copy(acc.at[pl.ds(row0, _RMAIN)],
                    out_hbm.at[c, pl.ds(row0, _RMAIN)])

    @pl.when(s == _NS - 1)
    def _():
        pltpu.sync_copy(acc.at[pl.ds(_NS * _RMAIN, _RREM)],
                        out_hbm.at[c, pl.ds(_NS * _RMAIN, _RREM)])


@functools.cache
def _sc_agg():
    return pl.kernel(
        _sc_agg_body,
        out_type=jax.ShapeDtypeStruct((_NC, _N, _D), jnp.float32),
        mesh=plsc.VectorSubcoreMesh(core_axis_name="c", subcore_axis_name="s",
                                    num_cores=_NC, num_subcores=_NS),
        scratch_types=[
            pltpu.VMEM((_NCHUNK, _CH), jnp.int32),
            pltpu.VMEM((_CH,), jnp.int32),
            pltpu.VMEM((_CH,), jnp.int32),
            pltpu.VMEM((_CH, _D), jnp.float32),
            pltpu.VMEM((_CH, _D), jnp.float32),
            pltpu.VMEM_SHARED((_N, _D), jnp.float32),
            pltpu.SemaphoreType.DMA,
            pltpu.SemaphoreType.DMA,
            pltpu.SemaphoreType.DMA,
            pltpu.SemaphoreType.DMA,
            pltpu.SemaphoreType.DMA,
        ],
    )


def kernel(x, edge_index, batch, fh_W1, fh_b1, fh_g1, fh_be1, fh_W2, fh_b2,
           fh_g2, fh_be2, nn_W1, nn_b1, nn_g1, nn_be1, nn_W2, nn_b2, nn_g2,
           nn_be2, lin0_W, lin0_b, lin1_W, lin1_b):
    batch_row = batch.reshape(1, _N)
    r = lambda v: v.reshape(1, -1)

    x1 = pl.pallas_call(
        _tc1_body,
        out_shape=jax.ShapeDtypeStruct((_N, _D), jnp.float32),
    )(x, fh_W1, r(fh_b1), r(fh_g1), r(fh_be1),
      fh_W2, r(fh_b2), r(fh_g2), r(fh_be2))

    edges4 = edge_index.reshape(2, _NW, _NCHUNK, _CH)
    aggp = _sc_agg()(edges4, x1)

    out0 = pl.pallas_call(
        _tcp_body,
        out_shape=jax.ShapeDtypeStruct((_G, _C), jnp.float32),
    )(x1, batch_row, lin0_W, r(lin0_b))

    out = pl.pallas_call(
        _tc2_body,
        out_shape=jax.ShapeDtypeStruct((_G, _C), jnp.float32),
    )(aggp, batch_row, out0, nn_W1, r(nn_b1), r(nn_g1), r(nn_be1),
      nn_W2, r(nn_b2), r(nn_g2), r(nn_be2), lin1_W, r(lin1_b))
    return out


# speedup vs baseline: 1.0071x; 1.0071x over previous
import functools

import jax
import jax.numpy as jnp
from jax import lax
from jax.experimental import pallas as pl
from jax.experimental.pallas import tpu as pltpu
from jax.experimental.pallas import tpu_sc as plsc

_N = 10000
_E = 320000
_D = 128
_G = 128
_C = 16

_NC = 2
_NS = 16
_NW = _NC * _NS
_CH = 128
_NCK = _E // _CH
_CPW = _NCK // _NW
_RMAIN = 624
_CPR = 48
_RREM = _N - _NS * _RMAIN


def _bn_relu(h, g, be):
    m = jnp.mean(h, axis=0, keepdims=True)
    v = jnp.mean((h - m) ** 2, axis=0, keepdims=True)
    return jnp.maximum((h - m) * lax.rsqrt(v + 1e-5) * g + be, 0.0)


def _mlp_block(h, W1, b1, g1, be1, W2, b2, g2, be2):
    h = jnp.dot(h, W1, preferred_element_type=jnp.float32) + b1
    h = _bn_relu(h, g1, be1)
    h = jnp.dot(h, W2, preferred_element_type=jnp.float32) + b2
    return _bn_relu(h, g2, be2)


def _pool_mats(batch_row):
    onehot = (batch_row == lax.broadcasted_iota(jnp.int32, (_G, _N), 0)
              ).astype(jnp.float32)
    counts = jnp.sum(onehot, axis=1, keepdims=True)
    return onehot, counts


def _tc1_body(x_ref, W1, b1, g1, be1, W2, b2, g2, be2, x1_out):
    x1_out[...] = _mlp_block(x_ref[...], W1[...], b1[...], g1[...], be1[...],
                             W2[...], b2[...], g2[...], be2[...])


def _tcp_body(x1_ref, batch_ref, l0W, l0b, out0_out):
    onehot, counts = _pool_mats(batch_ref[...])
    pooled = jnp.dot(onehot, x1_ref[...], preferred_element_type=jnp.float32)
    out0_out[...] = (jnp.dot(pooled, l0W[...], preferred_element_type=jnp.float32)
                     + counts * l0b[...])


def _tc2_body(agg_ref, batch_ref, out0_ref, W1, b1, g1, be1,
              W2, b2, g2, be2, l1W, l1b, out_ref):
    z = agg_ref[0] + agg_ref[1]
    x2 = _mlp_block(z, W1[...], b1[...], g1[...], be1[...],
                    W2[...], b2[...], g2[...], be2[...])
    onehot, counts = _pool_mats(batch_ref[...])
    pooled = jnp.dot(onehot, x2, preferred_element_type=jnp.float32)
    out = (out0_ref[...]
           + jnp.dot(pooled, l1W[...], preferred_element_type=jnp.float32)
           + counts * l1b[...])
    mx = jnp.max(out, axis=-1, keepdims=True)
    e = jnp.exp(out - mx)
    out_ref[...] = out - mx - jnp.log(jnp.sum(e, axis=-1, keepdims=True))


def _sc_agg_body(edges_hbm, x1_hbm, out_hbm,
                 sidx_a, didx_a, sidx_b, didx_b, rows_a, rows_b, acc,
                 sem_sa, sem_ta, sem_sb, sem_tb, sem_a, sem_b):
    c = lax.axis_index("c")
    s = lax.axis_index("s")
    w = s * _NC + c
    kb = w * _CPW

    pltpu.async_copy(edges_hbm.at[0, kb], sidx_a, sem_sa)
    pltpu.async_copy(edges_hbm.at[1, kb], didx_a, sem_ta)
    pltpu.async_copy(edges_hbm.at[0, kb + 1], sidx_b, sem_sb)
    pltpu.async_copy(edges_hbm.at[1, kb + 1], didx_b, sem_tb)

    row0 = s * _RMAIN

    @pl.when(c == 0)
    def _():
        pltpu.sync_copy(x1_hbm.at[pl.ds(row0, _RMAIN)],
                        acc.at[pl.ds(row0, _RMAIN)])

        @pl.when(s == _NS - 1)
        def _():
            pltpu.sync_copy(x1_hbm.at[pl.ds(_NS * _RMAIN, _RREM)],
                            acc.at[pl.ds(_NS * _RMAIN, _RREM)])

    @pl.when(c == 1)
    def _():
        def zstore(i, _):
            r = i // (_D // 16)
            col = (i % (_D // 16)) * 16
            rows_a[r, pl.ds(col, 16)] = jnp.zeros((16,), jnp.float32)
            return 0
        lax.fori_loop(0, _CPR * _D // 16, zstore, 0)
        for b in range(_RMAIN // _CPR):
            pltpu.sync_copy(rows_a.at[pl.ds(0, _CPR)],
                            acc.at[pl.ds(row0 + b * _CPR, _CPR)])

        @pl.when(s == _NS - 1)
        def _():
            pltpu.sync_copy(rows_a.at[pl.ds(0, _RREM)],
                            acc.at[pl.ds(_NS * _RMAIN, _RREM)])

    pltpu.make_async_copy(edges_hbm.at[0, kb], sidx_a, sem_sa).wait()
    pltpu.make_async_copy(edges_hbm.at[1, kb], didx_a, sem_ta).wait()
    pltpu.async_copy(x1_hbm.at[sidx_a], rows_a, sem_a)
    pltpu.make_async_copy(edges_hbm.at[0, kb + 1], sidx_b, sem_sb).wait()
    pltpu.make_async_copy(edges_hbm.at[1, kb + 1], didx_b, sem_tb).wait()
    pltpu.async_copy(x1_hbm.at[sidx_b], rows_b, sem_b)
    plsc.subcore_barrier()

    def half(j, sidx, didx, rows, sem_s, sem_t, sem_r):
        pltpu.make_async_copy(x1_hbm.at[sidx], rows, sem_r).wait()
        kn = kb + jnp.minimum(j + 2, _CPW - 1)
        pltpu.async_copy(edges_hbm.at[0, kn], sidx, sem_s)
        pltpu.sync_copy(rows, acc.at[didx], add=True)
        pltpu.async_copy(edges_hbm.at[1, kn], didx, sem_t)
        pltpu.make_async_copy(edges_hbm.at[0, kn], sidx, sem_s).wait()
        pltpu.make_async_copy(edges_hbm.at[1, kn], didx, sem_t).wait()
        pltpu.async_copy(x1_hbm.at[sidx], rows, sem_r)

    def step(i, _):
        j = 2 * i
        half(j, sidx_a, didx_a, rows_a, sem_sa, sem_ta, sem_a)
        half(j + 1, sidx_b, didx_b, rows_b, sem_sb, sem_tb, sem_b)
        return 0
    lax.fori_loop(0, _CPW // 2, step, 0)

    pltpu.make_async_copy(x1_hbm.at[sidx_a], rows_a, sem_a).wait()
    pltpu.make_async_copy(x1_hbm.at[sidx_b], rows_b, sem_b).wait()

    @pl.when(w < _NCK - _NW * _CPW)
    def _():
        ke = _NW * _CPW + w
        pltpu.sync_copy(edges_hbm.at[0, ke], sidx_a)
        pltpu.sync_copy(edges_hbm.at[1, ke], didx_a)
        pltpu.async_copy(x1_hbm.at[sidx_a], rows_a, sem_a).wait()
        pltpu.sync_copy(rows_a, acc.at[didx_a], add=True)
    plsc.subcore_barrier()

    pltpu.sync_copy(acc.at[pl.ds(row0, _RMAIN)],
                    out_hbm.at[c, pl.ds(row0, _RMAIN)])

    @pl.when(s == _NS - 1)
    def _():
        pltpu.sync_copy(acc.at[pl.ds(_NS * _RMAIN, _RREM)],
                        out_hbm.at[c, pl.ds(_NS * _RMAIN, _RREM)])


@functools.cache
def _sc_agg():
    return pl.kernel(
        _sc_agg_body,
        out_type=jax.ShapeDtypeStruct((_NC, _N, _D), jnp.float32),
        mesh=plsc.VectorSubcoreMesh(core_axis_name="c", subcore_axis_name="s",
                                    num_cores=_NC, num_subcores=_NS),
        scratch_types=[
            pltpu.VMEM((_CH,), jnp.int32),
            pltpu.VMEM((_CH,), jnp.int32),
            pltpu.VMEM((_CH,), jnp.int32),
            pltpu.VMEM((_CH,), jnp.int32),
            pltpu.VMEM((_CH, _D), jnp.float32),
            pltpu.VMEM((_CH, _D), jnp.float32),
            pltpu.VMEM_SHARED((_N, _D), jnp.float32),
            pltpu.SemaphoreType.DMA,
            pltpu.SemaphoreType.DMA,
            pltpu.SemaphoreType.DMA,
            pltpu.SemaphoreType.DMA,
            pltpu.SemaphoreType.DMA,
            pltpu.SemaphoreType.DMA,
        ],
    )


def kernel(x, edge_index, batch, fh_W1, fh_b1, fh_g1, fh_be1, fh_W2, fh_b2,
           fh_g2, fh_be2, nn_W1, nn_b1, nn_g1, nn_be1, nn_W2, nn_b2, nn_g2,
           nn_be2, lin0_W, lin0_b, lin1_W, lin1_b):
    batch_row = batch.reshape(1, _N)
    r = lambda v: v.reshape(1, -1)

    x1 = pl.pallas_call(
        _tc1_body,
        out_shape=jax.ShapeDtypeStruct((_N, _D), jnp.float32),
    )(x, fh_W1, r(fh_b1), r(fh_g1), r(fh_be1),
      fh_W2, r(fh_b2), r(fh_g2), r(fh_be2))

    edges3 = edge_index.reshape(2, _NCK, _CH)
    aggp = _sc_agg()(edges3, x1)

    out0 = pl.pallas_call(
        _tcp_body,
        out_shape=jax.ShapeDtypeStruct((_G, _C), jnp.float32),
    )(x1, batch_row, lin0_W, r(lin0_b))

    out = pl.pallas_call(
        _tc2_body,
        out_shape=jax.ShapeDtypeStruct((_G, _C), jnp.float32),
    )(aggp, batch_row, out0, nn_W1, r(nn_b1), r(nn_g1), r(nn_be1),
      nn_W2, r(nn_b2), r(nn_g2), r(nn_be2), lin1_W, r(lin1_b))
    return out


# speedup vs baseline: 1.1032x; 1.0954x over previous
import functools

import jax
import jax.numpy as jnp
from jax import lax
from jax.experimental import pallas as pl
from jax.experimental.pallas import tpu as pltpu
from jax.experimental.pallas import tpu_sc as plsc

_N = 10000
_E = 320000
_D = 128
_G = 128
_C = 16

_NC = 2
_NS = 16
_NW = _NC * _NS
_CH = 128
_NCK = _E // _CH
_CPW = _NCK // _NW
_RMAIN = 624
_CPR = 48
_RREM = _N - _NS * _RMAIN


def _bn_relu(h, g, be):
    m = jnp.mean(h, axis=0, keepdims=True)
    v = jnp.mean((h - m) ** 2, axis=0, keepdims=True)
    return jnp.maximum((h - m) * lax.rsqrt(v + 1e-5) * g + be, 0.0)


def _mlp_block(h, W1, b1, g1, be1, W2, b2, g2, be2):
    h = jnp.dot(h, W1, preferred_element_type=jnp.float32) + b1
    h = _bn_relu(h, g1, be1)
    h = jnp.dot(h, W2, preferred_element_type=jnp.float32) + b2
    return _bn_relu(h, g2, be2)


def _pool_mats(batch_row):
    onehot = (batch_row == lax.broadcasted_iota(jnp.int32, (_G, _N), 0)
              ).astype(jnp.float32)
    counts = jnp.sum(onehot, axis=1, keepdims=True)
    return onehot, counts


def _tc1_body(x_ref, W1, b1, g1, be1, W2, b2, g2, be2, x1_out):
    x1_out[...] = _mlp_block(x_ref[...], W1[...], b1[...], g1[...], be1[...],
                             W2[...], b2[...], g2[...], be2[...])


def _tcp_body(x1_ref, batch_ref, l0W, l0b, out0_out):
    onehot, counts = _pool_mats(batch_ref[...])
    pooled = jnp.dot(onehot, x1_ref[...], preferred_element_type=jnp.float32)
    out0_out[...] = (jnp.dot(pooled, l0W[...], preferred_element_type=jnp.float32)
                     + counts * l0b[...])


def _tc2_body(agg_ref, batch_ref, out0_ref, W1, b1, g1, be1,
              W2, b2, g2, be2, l1W, l1b, out_ref):
    z = agg_ref[0] + agg_ref[1]
    x2 = _mlp_block(z, W1[...], b1[...], g1[...], be1[...],
                    W2[...], b2[...], g2[...], be2[...])
    onehot, counts = _pool_mats(batch_ref[...])
    pooled = jnp.dot(onehot, x2, preferred_element_type=jnp.float32)
    out = (out0_ref[...]
           + jnp.dot(pooled, l1W[...], preferred_element_type=jnp.float32)
           + counts * l1b[...])
    mx = jnp.max(out, axis=-1, keepdims=True)
    e = jnp.exp(out - mx)
    out_ref[...] = out - mx - jnp.log(jnp.sum(e, axis=-1, keepdims=True))


def _sc_agg_body(edges_hbm, x1_hbm, out_hbm,
                 sidx_a, didx_a, sidx_b, didx_b, rows_a, rows_b, acc,
                 sem_sa, sem_ta, sem_sb, sem_tb, sem_a, sem_b):
    c = lax.axis_index("c")
    s = lax.axis_index("s")
    w = s * _NC + c
    kb = w * _CPW

    pltpu.async_copy(edges_hbm.at[0, kb], sidx_a, sem_sa)
    pltpu.async_copy(edges_hbm.at[1, kb], didx_a, sem_ta)
    pltpu.async_copy(edges_hbm.at[0, kb + 1], sidx_b, sem_sb)
    pltpu.async_copy(edges_hbm.at[1, kb + 1], didx_b, sem_tb)

    row0 = s * _RMAIN

    @pl.when(c == 0)
    def _():
        pltpu.sync_copy(x1_hbm.at[pl.ds(row0, _RMAIN)],
                        acc.at[pl.ds(row0, _RMAIN)])

        @pl.when(s == _NS - 1)
        def _():
            pltpu.sync_copy(x1_hbm.at[pl.ds(_NS * _RMAIN, _RREM)],
                            acc.at[pl.ds(_NS * _RMAIN, _RREM)])

    @pl.when(c == 1)
    def _():
        def zstore(i, _):
            r = i // (_D // 16)
            col = (i % (_D // 16)) * 16
            rows_a[r, pl.ds(col, 16)] = jnp.zeros((16,), jnp.float32)
            return 0
        lax.fori_loop(0, _CPR * _D // 16, zstore, 0)
        for b in range(_RMAIN // _CPR):
            pltpu.sync_copy(rows_a.at[pl.ds(0, _CPR)],
                            acc.at[pl.ds(row0 + b * _CPR, _CPR)])

        @pl.when(s == _NS - 1)
        def _():
            pltpu.sync_copy(rows_a.at[pl.ds(0, _RREM)],
                            acc.at[pl.ds(_NS * _RMAIN, _RREM)])

    pltpu.make_async_copy(edges_hbm.at[0, kb], sidx_a, sem_sa).wait()
    pltpu.async_copy(x1_hbm.at[sidx_a], rows_a, sem_a)
    pltpu.make_async_copy(edges_hbm.at[0, kb + 1], sidx_b, sem_sb).wait()
    pltpu.async_copy(x1_hbm.at[sidx_b], rows_b, sem_b)
    plsc.subcore_barrier()

    def half(j, sidx, didx, rows, sem_s, sem_t, sem_r):
        pltpu.make_async_copy(x1_hbm.at[sidx], rows, sem_r).wait()
        kn = kb + jnp.minimum(j + 2, _CPW - 1)
        pltpu.async_copy(edges_hbm.at[0, kn], sidx, sem_s)
        pltpu.make_async_copy(edges_hbm.at[1, kn], didx, sem_t).wait()
        pltpu.sync_copy(rows, acc.at[didx], add=True)
        pltpu.async_copy(edges_hbm.at[1, kn], didx, sem_t)
        pltpu.make_async_copy(edges_hbm.at[0, kn], sidx, sem_s).wait()
        pltpu.async_copy(x1_hbm.at[sidx], rows, sem_r)

    def step(i, _):
        j = 2 * i
        half(j, sidx_a, didx_a, rows_a, sem_sa, sem_ta, sem_a)
        half(j + 1, sidx_b, didx_b, rows_b, sem_sb, sem_tb, sem_b)
        return 0
    lax.fori_loop(0, _CPW // 2, step, 0)

    pltpu.make_async_copy(x1_hbm.at[sidx_a], rows_a, sem_a).wait()
    pltpu.make_async_copy(x1_hbm.at[sidx_b], rows_b, sem_b).wait()
    pltpu.make_async_copy(edges_hbm.at[1, kb], didx_a, sem_ta).wait()
    pltpu.make_async_copy(edges_hbm.at[1, kb], didx_b, sem_tb).wait()

    @pl.when(w < _NCK - _NW * _CPW)
    def _():
        ke = _NW * _CPW + w
        pltpu.sync_copy(edges_hbm.at[0, ke], sidx_a)
        pltpu.sync_copy(edges_hbm.at[1, ke], didx_a)
        pltpu.async_copy(x1_hbm.at[sidx_a], rows_a, sem_a).wait()
        pltpu.sync_copy(rows_a, acc.at[didx_a], add=True)
    plsc.subcore_barrier()

    pltpu.sync_copy(acc.at[pl.ds(row0, _RMAIN)],
                    out_hbm.at[c, pl.ds(row0, _RMAIN)])

    @pl.when(s == _NS - 1)
    def _():
        pltpu.sync_copy(acc.at[pl.ds(_NS * _RMAIN, _RREM)],
                        out_hbm.at[c, pl.ds(_NS * _RMAIN, _RREM)])


@functools.cache
def _sc_agg():
    return pl.kernel(
        _sc_agg_body,
        out_type=jax.ShapeDtypeStruct((_NC, _N, _D), jnp.float32),
        mesh=plsc.VectorSubcoreMesh(core_axis_name="c", subcore_axis_name="s",
                                    num_cores=_NC, num_subcores=_NS),
        scratch_types=[
            pltpu.VMEM((_CH,), jnp.int32),
            pltpu.VMEM((_CH,), jnp.int32),
            pltpu.VMEM((_CH,), jnp.int32),
            pltpu.VMEM((_CH,), jnp.int32),
            pltpu.VMEM((_CH, _D), jnp.float32),
            pltpu.VMEM((_CH, _D), jnp.float32),
            pltpu.VMEM_SHARED((_N, _D), jnp.float32),
            pltpu.SemaphoreType.DMA,
            pltpu.SemaphoreType.DMA,
            pltpu.SemaphoreType.DMA,
            pltpu.SemaphoreType.DMA,
            pltpu.SemaphoreType.DMA,
            pltpu.SemaphoreType.DMA,
        ],
    )


def kernel(x, edge_index, batch, fh_W1, fh_b1, fh_g1, fh_be1, fh_W2, fh_b2,
           fh_g2, fh_be2, nn_W1, nn_b1, nn_g1, nn_be1, nn_W2, nn_b2, nn_g2,
           nn_be2, lin0_W, lin0_b, lin1_W, lin1_b):
    batch_row = batch.reshape(1, _N)
    r = lambda v: v.reshape(1, -1)

    x1 = pl.pallas_call(
        _tc1_body,
        out_shape=jax.ShapeDtypeStruct((_N, _D), jnp.float32),
    )(x, fh_W1, r(fh_b1), r(fh_g1), r(fh_be1),
      fh_W2, r(fh_b2), r(fh_g2), r(fh_be2))

    edges3 = edge_index.reshape(2, _NCK, _CH)
    aggp = _sc_agg()(edges3, x1)

    out0 = pl.pallas_call(
        _tcp_body,
        out_shape=jax.ShapeDtypeStruct((_G, _C), jnp.float32),
    )(x1, batch_row, lin0_W, r(lin0_b))

    out = pl.pallas_call(
        _tc2_body,
        out_shape=jax.ShapeDtypeStruct((_G, _C), jnp.float32),
    )(aggp, batch_row, out0, nn_W1, r(nn_b1), r(nn_g1), r(nn_be1),
      nn_W2, r(nn_b2), r(nn_g2), r(nn_be2), lin1_W, r(lin1_b))
    return out
